# Initial kernel scaffold; baseline (speedup 1.0000x reference)
import jax
import jax.numpy as jnp
from jax.experimental import pallas as pl


def kernel(loc_data, conf_data, landm_data, priors, targets):
    raise NotImplementedError("write your pallas kernel here")


# speedup vs baseline: 5.8744x; 5.8744x over previous
import functools

import jax
import jax.numpy as jnp
import numpy as np
from jax.experimental import pallas as pl
from jax.experimental.pallas import tpu as pltpu

_NUM_CLASSES = 2
_P_TH = np.float32(0.35)
_P_TH2 = np.float32(0.35 + 0.05)
_N_TH = np.float32(0.35)
_K_NUM = 10
_NEGPOS = 7.0
_V0 = np.float32(0.1)
_V1 = np.float32(0.2)

_LANES = 128


def _sl1(d):
    return jnp.where(d < 1.0, 0.5 * d * d, d - 0.5)


def _image_body(nobj, rows, num_anchor, loc_ref, conf_ref, landm_ref, pri_ref,
                targ_ref, out_ref):
    f32 = jnp.float32
    shp = (rows, _LANES)
    gidx = (jax.lax.broadcasted_iota(jnp.int32, shp, 0) * _LANES
            + jax.lax.broadcasted_iota(jnp.int32, shp, 1))
    validm = gidx < num_anchor

    pcx = pri_ref[0]
    pcy = pri_ref[1]
    pw = pri_ref[2]
    ph = pri_ref[3]
    px1 = pcx - pw / 2.0
    py1 = pcy - ph / 2.0
    px2 = pcx + pw / 2.0
    py2 = pcy + ph / 2.0
    area_p = (px2 - px1) * (py2 - py1)

    l0 = loc_ref[0, 0]
    l1 = loc_ref[0, 1]
    l2 = loc_ref[0, 2]
    l3 = loc_ref[0, 3]
    dcx = pcx + l0 * _V0 * pw
    dcy = pcy + l1 * _V0 * ph
    dw = pw * jnp.exp(l2 * _V1)
    dh = ph * jnp.exp(l3 * _V1)
    dx1 = dcx - dw / 2.0
    dy1 = dcy - dh / 2.0
    dx2 = dcx + dw / 2.0
    dy2 = dcy + dh / 2.0
    area_d = (dx2 - dx1) * (dy2 - dy1)

    big_idx = jnp.int32(rows * _LANES)

    def topk_found(row):
        def body(k, carry):
            work, found, first = carry
            m = jnp.max(work)
            cand = jnp.where(work == m, gidx, big_idx)
            i0 = jnp.min(cand)
            oneh = (gidx == i0).astype(f32)
            found = jnp.maximum(found, oneh)
            first = jnp.where(k == 0, i0, first)
            work = jnp.where(oneh > 0, f32(-5.0), work)
            return work, found, first

        init = (row, jnp.zeros(shp, f32), jnp.int32(0))
        _, found, first = jax.lax.fori_loop(0, _K_NUM, body, init)
        return found > 0, first

    killed = jnp.zeros(shp, jnp.bool_)
    maxrow = jnp.full(shp, -1e30, f32)
    loss_l = f32(0.0)
    loss_lm = f32(0.0)
    nval = f32(0.0)
    n1val = f32(0.0)

    for i in range(nobj):
        tx1 = targ_ref[0, i, 0]
        ty1 = targ_ref[0, i, 1]
        tx2 = targ_ref[0, i, 2]
        ty2 = targ_ref[0, i, 3]
        area_t = (tx2 - tx1) * (ty2 - ty1)

        iw = jnp.clip(jnp.minimum(tx2, px2) - jnp.maximum(tx1, px1), 0.0, None)
        ih = jnp.clip(jnp.minimum(ty2, py2) - jnp.maximum(ty1, py1), 0.0, None)
        inter_a = iw * ih
        iou_a = inter_a / (area_t + area_p - inter_a)

        iwp = jnp.clip(jnp.minimum(tx2, dx2) - jnp.maximum(tx1, dx1), 0.0, None)
        ihp = jnp.clip(jnp.minimum(ty2, dy2) - jnp.maximum(ty1, dy1), 0.0, None)
        inter_p = iwp * ihp
        iou_p = inter_p / (area_t + area_d - inter_p)

        maxrow = jnp.maximum(maxrow, jnp.maximum(iou_a, iou_p))

        rowa = jnp.where(killed, f32(-1.0), iou_a)
        rowa = jnp.where(validm, rowa, f32(-3.0))
        founda, firsta = topk_found(rowa)
        passa = jnp.where(founda & (rowa > _P_TH), f32(1.0), f32(0.0))
        anyg = jnp.max(passa) > 0
        firsta_f = jnp.where(gidx == firsta, f32(1.0), f32(0.0))
        hit_a = jnp.where(anyg, passa, firsta_f) > 0
        killed = killed | hit_a

        rowp = jnp.where(killed, f32(-1.0), iou_p)
        rowp = jnp.where(validm, rowp, f32(-3.0))
        foundp, _ = topk_found(rowp)
        hit_p = foundp & (rowp > _P_TH2)
        killed = killed | hit_p

        mi = hit_a | hit_p

        g_cx = ((tx1 + tx2) / 2.0 - pcx) / (_V0 * pw)
        g_cy = ((ty1 + ty2) / 2.0 - pcy) / (_V0 * ph)
        g_w = jnp.log((tx2 - tx1) / pw) / _V1
        g_h = jnp.log((ty2 - ty1) / ph) / _V1
        lrow = (_sl1(jnp.abs(l0 - g_cx)) + _sl1(jnp.abs(l1 - g_cy))
                + _sl1(jnp.abs(l2 - g_w)) + _sl1(jnp.abs(l3 - g_h)))
        loss_l = loss_l + jnp.sum(jnp.where(mi, lrow, f32(0.0)))

        lmrow = jnp.zeros(shp, f32)
        for j in range(3):
            lmx = targ_ref[0, i, 4 + 2 * j]
            lmy = targ_ref[0, i, 5 + 2 * j]
            gx = (lmx - pcx) / (_V0 * pw)
            gy = (lmy - pcy) / (_V0 * ph)
            lmrow = lmrow + _sl1(jnp.abs(landm_ref[0, 2 * j] - gx))
            lmrow = lmrow + _sl1(jnp.abs(landm_ref[0, 2 * j + 1] - gy))
        lab = targ_ref[0, i, 14]
        labf = jnp.where(lab == 1.0, f32(1.0), f32(0.0))
        loss_lm = loss_lm + labf * jnp.sum(jnp.where(mi, lmrow, f32(0.0)))

        cnt_i = jnp.sum(mi.astype(f32))
        nval = nval + cnt_i
        n1val = n1val + labf * cnt_i

    sel = validm & (~killed) & (maxrow < _N_TH)
    c0 = conf_ref[0, 0]
    c1 = conf_ref[0, 1]
    m01 = jnp.maximum(c0, c1)
    logz = m01 + jnp.log(jnp.exp(c0 - m01) + jnp.exp(c1 - m01))
    dcf = c1 - c0
    negval = logz - c0

    num_pos = jnp.sum(jnp.where(killed, f32(1.0), f32(0.0)))
    cnt_sel = jnp.sum(jnp.where(sel, f32(1.0), f32(0.0)))
    t_eff = jnp.minimum(num_pos * _NEGPOS, cnt_sel)

    s_all = jnp.sum(jnp.where(sel, negval, f32(0.0)))

    hi0 = jnp.max(jnp.where(sel, dcf, f32(-1e30)))
    lo0 = jnp.min(jnp.where(sel, dcf, f32(1e30))) - 1.0

    def bbody(_, carry):
        lo, hi = carry
        mid = 0.5 * (lo + hi)
        c = jnp.sum(jnp.where(sel & (dcf > mid), f32(1.0), f32(0.0)))
        ge = c >= t_eff
        return jnp.where(ge, mid, lo), jnp.where(ge, hi, mid)

    lo, hi = jax.lax.fori_loop(0, 48, bbody, (lo0, hi0))
    above = sel & (dcf > hi)
    g_cnt = jnp.sum(jnp.where(above, f32(1.0), f32(0.0)))
    s1 = jnp.sum(jnp.where(above, negval, f32(0.0)))
    bnd = sel & (dcf <= hi) & (dcf > lo)
    cnt2 = jnp.sum(jnp.where(bnd, f32(1.0), f32(0.0)))
    s2 = jnp.sum(jnp.where(bnd, negval, f32(0.0)))
    s_bis = s1 + (t_eff - g_cnt) * (s2 / jnp.maximum(cnt2, f32(1.0)))
    s_neg = jnp.where(t_eff >= cnt_sel, s_all, s_bis)

    pos_part = jnp.sum(jnp.where(killed, logz - c1, f32(0.0)))
    loss_cls = pos_part + s_neg

    lane = jax.lax.broadcasted_iota(jnp.int32, (1, _LANES), 1)
    vec = jnp.where(lane == 0, loss_l,
                    jnp.where(lane == 1, loss_cls,
                              jnp.where(lane == 2, loss_lm,
                                        jnp.where(lane == 3, nval,
                                                  jnp.where(lane == 4, n1val,
                                                            f32(0.0))))))
    out_ref[0] = vec


@jax.jit
def kernel(loc_data, conf_data, landm_data, priors, targets):
    num, num_anchor, _ = loc_data.shape
    nobj = targets.shape[1]
    rows = (num_anchor + _LANES - 1) // _LANES
    rows = ((rows + 7) // 8) * 8
    a_pad = rows * _LANES
    pad = a_pad - num_anchor

    def prep(x):
        x = jnp.pad(x, ((0, 0), (0, pad), (0, 0)))
        return x.transpose(0, 2, 1).reshape(num, x.shape[2], rows, _LANES)

    loc_r = prep(loc_data)
    conf_r = prep(conf_data)
    landm_r = prep(landm_data)
    pri_r = jnp.pad(priors, ((0, pad), (0, 0))).T.reshape(4, rows, _LANES)
    targ = jnp.pad(targets, ((0, 0), (0, 0), (0, 1)))

    body = functools.partial(_image_body, nobj, rows, num_anchor)
    out = pl.pallas_call(
        body,
        grid=(num,),
        in_specs=[
            pl.BlockSpec((1, 4, rows, _LANES), lambda b: (b, 0, 0, 0)),
            pl.BlockSpec((1, _NUM_CLASSES, rows, _LANES),
                         lambda b: (b, 0, 0, 0)),
            pl.BlockSpec((1, 6, rows, _LANES), lambda b: (b, 0, 0, 0)),
            pl.BlockSpec((4, rows, _LANES), lambda b: (0, 0, 0)),
            pl.BlockSpec((1, nobj, 16), lambda b: (b, 0, 0),
                         memory_space=pltpu.SMEM),
        ],
        out_specs=pl.BlockSpec((1, 1, _LANES), lambda b: (b, 0, 0)),
        out_shape=jax.ShapeDtypeStruct((num, 1, _LANES), jnp.float32),
    )(loc_r, conf_r, landm_r, pri_r, targ)

    sums = jnp.sum(out[:, 0, :5], axis=0)
    return (sums[0] / sums[3], sums[1] / sums[3], sums[2] / sums[4])


# speedup vs baseline: 10.6699x; 1.8163x over previous
import functools

import jax
import jax.numpy as jnp
import numpy as np
from jax.experimental import pallas as pl
from jax.experimental.pallas import tpu as pltpu

_NUM_CLASSES = 2
_P_TH = np.float32(0.35)
_P_TH2 = np.float32(0.35 + 0.05)
_N_TH = np.float32(0.35)
_K_NUM = 10
_NEGPOS = 7.0
_V0 = np.float32(0.1)
_V1 = np.float32(0.2)

_LANES = 128
_NIMG = 2
_BISECT_ITERS = 40


def _sl1(d):
    return jnp.where(d < 1.0, 0.5 * d * d, d - 0.5)


def _image_body(nobj, rows, num_anchor, loc_ref, conf_ref, landm_ref, pri_ref,
                targ_ref, out_ref):
    f32 = jnp.float32
    shp = (rows, _LANES)
    gidx = (jax.lax.broadcasted_iota(jnp.int32, shp, 0) * _LANES
            + jax.lax.broadcasted_iota(jnp.int32, shp, 1))
    validm = gidx < num_anchor
    big_idx = jnp.int32(rows * _LANES)

    pcx = pri_ref[0]
    pcy = pri_ref[1]
    pw = pri_ref[2]
    ph = pri_ref[3]
    px1 = pcx - pw / 2.0
    py1 = pcy - ph / 2.0
    px2 = pcx + pw / 2.0
    py2 = pcy + ph / 2.0
    area_p = (px2 - px1) * (py2 - py1)

    n = _NIMG
    l0 = [loc_ref[j, 0] for j in range(n)]
    l1 = [loc_ref[j, 1] for j in range(n)]
    l2 = [loc_ref[j, 2] for j in range(n)]
    l3 = [loc_ref[j, 3] for j in range(n)]
    dx1, dy1, dx2, dy2, area_d = [], [], [], [], []
    for j in range(n):
        dcx = pcx + l0[j] * _V0 * pw
        dcy = pcy + l1[j] * _V0 * ph
        dw = pw * jnp.exp(l2[j] * _V1)
        dh = ph * jnp.exp(l3[j] * _V1)
        dx1.append(dcx - dw / 2.0)
        dy1.append(dcy - dh / 2.0)
        dx2.append(dcx + dw / 2.0)
        dy2.append(dcy + dh / 2.0)
        area_d.append((dx2[j] - dx1[j]) * (dy2[j] - dy1[j]))

    def topk_found(rowlist, need_first):
        def body(k, carry):
            works, firsts = carry
            ms = [jnp.max(w) for w in works]
            i0s = [jnp.min(jnp.where(w == m, gidx, big_idx))
                   for w, m in zip(works, ms)]
            if need_first:
                firsts = tuple(jnp.where(k == 0, i0, fr)
                               for i0, fr in zip(i0s, firsts))
            works = tuple(jnp.where(gidx == i0, f32(-5.0), w)
                          for i0, w in zip(i0s, works))
            return works, firsts

        init = (tuple(rowlist), tuple(jnp.int32(0) for _ in rowlist))
        works, firsts = jax.lax.fori_loop(0, _K_NUM, body, init)
        founds = [w != r for w, r in zip(works, rowlist)]
        return founds, firsts

    killed = [jnp.zeros(shp, jnp.bool_) for _ in range(n)]
    maxrow = [jnp.full(shp, -1e30, f32) for _ in range(n)]
    loss_l = [f32(0.0)] * n
    loss_lm = [f32(0.0)] * n
    nval = [f32(0.0)] * n
    n1val = [f32(0.0)] * n

    for i in range(nobj):
        tx1 = [targ_ref[j, i, 0] for j in range(n)]
        ty1 = [targ_ref[j, i, 1] for j in range(n)]
        tx2 = [targ_ref[j, i, 2] for j in range(n)]
        ty2 = [targ_ref[j, i, 3] for j in range(n)]
        area_t = [(tx2[j] - tx1[j]) * (ty2[j] - ty1[j]) for j in range(n)]

        iou_a, iou_p, rowa = [], [], []
        for j in range(n):
            iw = jnp.clip(jnp.minimum(tx2[j], px2) - jnp.maximum(tx1[j], px1),
                          0.0, None)
            ih = jnp.clip(jnp.minimum(ty2[j], py2) - jnp.maximum(ty1[j], py1),
                          0.0, None)
            inter_a = iw * ih
            ia = inter_a / (area_t[j] + area_p - inter_a)
            iou_a.append(ia)

            iwp = jnp.clip(
                jnp.minimum(tx2[j], dx2[j]) - jnp.maximum(tx1[j], dx1[j]),
                0.0, None)
            ihp = jnp.clip(
                jnp.minimum(ty2[j], dy2[j]) - jnp.maximum(ty1[j], dy1[j]),
                0.0, None)
            inter_p = iwp * ihp
            ip = inter_p / (area_t[j] + area_d[j] - inter_p)
            iou_p.append(ip)

            maxrow[j] = jnp.maximum(maxrow[j], jnp.maximum(ia, ip))
            ra = jnp.where(killed[j], f32(-1.0), ia)
            rowa.append(jnp.where(validm, ra, f32(-3.0)))

        founda, firsta = topk_found(rowa, True)
        hit_a, rowp = [], []
        for j in range(n):
            passa = jnp.where(founda[j] & (rowa[j] > _P_TH), f32(1.0),
                              f32(0.0))
            anyg = jnp.max(passa) > 0
            firsta_f = jnp.where(gidx == firsta[j], f32(1.0), f32(0.0))
            ha = jnp.where(anyg, passa, firsta_f) > 0
            hit_a.append(ha)
            killed[j] = killed[j] | ha
            rp = jnp.where(killed[j], f32(-1.0), iou_p[j])
            rowp.append(jnp.where(validm, rp, f32(-3.0)))

        foundp, _ = topk_found(rowp, False)
        for j in range(n):
            hit_p = foundp[j] & (rowp[j] > _P_TH2)
            killed[j] = killed[j] | hit_p
            mi = hit_a[j] | hit_p

            g_cx = ((tx1[j] + tx2[j]) / 2.0 - pcx) / (_V0 * pw)
            g_cy = ((ty1[j] + ty2[j]) / 2.0 - pcy) / (_V0 * ph)
            g_w = jnp.log((tx2[j] - tx1[j]) / pw) / _V1
            g_h = jnp.log((ty2[j] - ty1[j]) / ph) / _V1
            lrow = (_sl1(jnp.abs(l0[j] - g_cx)) + _sl1(jnp.abs(l1[j] - g_cy))
                    + _sl1(jnp.abs(l2[j] - g_w)) + _sl1(jnp.abs(l3[j] - g_h)))
            loss_l[j] = loss_l[j] + jnp.sum(jnp.where(mi, lrow, f32(0.0)))

            lmrow = jnp.zeros(shp, f32)
            for p in range(3):
                lmx = targ_ref[j, i, 4 + 2 * p]
                lmy = targ_ref[j, i, 5 + 2 * p]
                gx = (lmx - pcx) / (_V0 * pw)
                gy = (lmy - pcy) / (_V0 * ph)
                lmrow = lmrow + _sl1(jnp.abs(landm_ref[j, 2 * p] - gx))
                lmrow = lmrow + _sl1(jnp.abs(landm_ref[j, 2 * p + 1] - gy))
            lab = targ_ref[j, i, 14]
            labf = jnp.where(lab == 1.0, f32(1.0), f32(0.0))
            loss_lm[j] = loss_lm[j] + labf * jnp.sum(
                jnp.where(mi, lmrow, f32(0.0)))

            cnt_i = jnp.sum(mi.astype(f32))
            nval[j] = nval[j] + cnt_i
            n1val[j] = n1val[j] + labf * cnt_i

    sel, dcf, negval, logz, t_eff, cnt_sel = [], [], [], [], [], []
    for j in range(n):
        sel.append(validm & (~killed[j]) & (maxrow[j] < _N_TH))
        c0 = conf_ref[j, 0]
        c1 = conf_ref[j, 1]
        m01 = jnp.maximum(c0, c1)
        lz = m01 + jnp.log(jnp.exp(c0 - m01) + jnp.exp(c1 - m01))
        logz.append(lz)
        dcf.append(c1 - c0)
        negval.append(lz - c0)
        num_pos = jnp.sum(jnp.where(killed[j], f32(1.0), f32(0.0)))
        cs = jnp.sum(jnp.where(sel[j], f32(1.0), f32(0.0)))
        cnt_sel.append(cs)
        t_eff.append(jnp.minimum(num_pos * _NEGPOS, cs))

    his = tuple(jnp.max(jnp.where(sel[j], dcf[j], f32(-1e30)))
                for j in range(n))
    los = tuple(jnp.min(jnp.where(sel[j], dcf[j], f32(1e30))) - 1.0
                for j in range(n))

    def bbody(_, carry):
        los, his = carry
        nlo, nhi = [], []
        for j in range(n):
            mid = 0.5 * (los[j] + his[j])
            c = jnp.sum(jnp.where(sel[j] & (dcf[j] > mid), f32(1.0), f32(0.0)))
            ge = c >= t_eff[j]
            nlo.append(jnp.where(ge, mid, los[j]))
            nhi.append(jnp.where(ge, his[j], mid))
        return tuple(nlo), tuple(nhi)

    los, his = jax.lax.fori_loop(0, _BISECT_ITERS, bbody, (los, his))

    for j in range(n):
        s_all = jnp.sum(jnp.where(sel[j], negval[j], f32(0.0)))
        above = sel[j] & (dcf[j] > his[j])
        g_cnt = jnp.sum(jnp.where(above, f32(1.0), f32(0.0)))
        s1 = jnp.sum(jnp.where(above, negval[j], f32(0.0)))
        bnd = sel[j] & (dcf[j] <= his[j]) & (dcf[j] > los[j])
        cnt2 = jnp.sum(jnp.where(bnd, f32(1.0), f32(0.0)))
        s2 = jnp.sum(jnp.where(bnd, negval[j], f32(0.0)))
        s_bis = s1 + (t_eff[j] - g_cnt) * (s2 / jnp.maximum(cnt2, f32(1.0)))
        s_neg = jnp.where(t_eff[j] >= cnt_sel[j], s_all, s_bis)
        pos_part = jnp.sum(jnp.where(killed[j], logz[j] - conf_ref[j, 1],
                                     f32(0.0)))
        loss_cls = pos_part + s_neg

        lane = jax.lax.broadcasted_iota(jnp.int32, (1, _LANES), 1)
        vec = jnp.where(
            lane == 0, loss_l[j],
            jnp.where(lane == 1, loss_cls,
                      jnp.where(lane == 2, loss_lm[j],
                                jnp.where(lane == 3, nval[j],
                                          jnp.where(lane == 4, n1val[j],
                                                    f32(0.0))))))
        out_ref[j] = vec


@jax.jit
def kernel(loc_data, conf_data, landm_data, priors, targets):
    num, num_anchor, _ = loc_data.shape
    nobj = targets.shape[1]
    rows = (num_anchor + _LANES - 1) // _LANES
    rows = ((rows + 7) // 8) * 8
    a_pad = rows * _LANES
    pad = a_pad - num_anchor

    def prep(x):
        x = jnp.pad(x, ((0, 0), (0, pad), (0, 0)))
        return x.transpose(0, 2, 1).reshape(num, x.shape[2], rows, _LANES)

    loc_r = prep(loc_data)
    conf_r = prep(conf_data)
    landm_r = prep(landm_data)
    pri_r = jnp.pad(priors, ((0, pad), (0, 0))).T.reshape(4, rows, _LANES)
    targ = jnp.pad(targets, ((0, 0), (0, 0), (0, 1)))

    body = functools.partial(_image_body, nobj, rows, num_anchor)
    out = pl.pallas_call(
        body,
        grid=(num // _NIMG,),
        in_specs=[
            pl.BlockSpec((_NIMG, 4, rows, _LANES), lambda b: (b, 0, 0, 0)),
            pl.BlockSpec((_NIMG, _NUM_CLASSES, rows, _LANES),
                         lambda b: (b, 0, 0, 0)),
            pl.BlockSpec((_NIMG, 6, rows, _LANES), lambda b: (b, 0, 0, 0)),
            pl.BlockSpec((4, rows, _LANES), lambda b: (0, 0, 0)),
            pl.BlockSpec((_NIMG, nobj, 16), lambda b: (b, 0, 0),
                         memory_space=pltpu.SMEM),
        ],
        out_specs=pl.BlockSpec((_NIMG, 1, _LANES), lambda b: (b, 0, 0)),
        out_shape=jax.ShapeDtypeStruct((num, 1, _LANES), jnp.float32),
    )(loc_r, conf_r, landm_r, pri_r, targ)

    sums = jnp.sum(out[:, 0, :5], axis=0)
    return (sums[0] / sums[3], sums[1] / sums[3], sums[2] / sums[4])


# speedup vs baseline: 16.0947x; 1.5084x over previous
import functools

import jax
import jax.numpy as jnp
import numpy as np
from jax.experimental import pallas as pl
from jax.experimental.pallas import tpu as pltpu

_NUM_CLASSES = 2
_P_TH = np.float32(0.35)
_P_TH2 = np.float32(0.35 + 0.05)
_N_TH = np.float32(0.35)
_K_NUM = 10
_NEGPOS = 7.0
_V0 = np.float32(0.1)
_V1 = np.float32(0.2)

_LANES = 128
_NIMG = 4
_BISECT_ITERS = 40


def _sl1(d):
    return jnp.where(d < 1.0, 0.5 * d * d, d - 0.5)


def _image_body(nobj, rows, num_anchor, loc_ref, conf_ref, landm_ref, pri_ref,
                targ_ref, out_ref):
    f32 = jnp.float32
    shp = (rows, _LANES)
    gidx = (jax.lax.broadcasted_iota(jnp.int32, shp, 0) * _LANES
            + jax.lax.broadcasted_iota(jnp.int32, shp, 1))
    validm = gidx < num_anchor
    big_idx = jnp.int32(rows * _LANES)

    pcx = pri_ref[0]
    pcy = pri_ref[1]
    pw = pri_ref[2]
    ph = pri_ref[3]
    px1 = pcx - pw / 2.0
    py1 = pcy - ph / 2.0
    px2 = pcx + pw / 2.0
    py2 = pcy + ph / 2.0
    area_p = (px2 - px1) * (py2 - py1)

    n = _NIMG
    l0 = [loc_ref[j, 0] for j in range(n)]
    l1 = [loc_ref[j, 1] for j in range(n)]
    l2 = [loc_ref[j, 2] for j in range(n)]
    l3 = [loc_ref[j, 3] for j in range(n)]
    dx1, dy1, dx2, dy2, area_d = [], [], [], [], []
    for j in range(n):
        dcx = pcx + l0[j] * _V0 * pw
        dcy = pcy + l1[j] * _V0 * ph
        dw = pw * jnp.exp(l2[j] * _V1)
        dh = ph * jnp.exp(l3[j] * _V1)
        dx1.append(dcx - dw / 2.0)
        dy1.append(dcy - dh / 2.0)
        dx2.append(dcx + dw / 2.0)
        dy2.append(dcy + dh / 2.0)
        area_d.append((dx2[j] - dx1[j]) * (dy2[j] - dy1[j]))

    def topk_found(rowlist, need_first):
        def body(k, carry):
            works, firsts = carry
            ms = [jnp.max(w) for w in works]
            i0s = [jnp.min(jnp.where(w == m, gidx, big_idx))
                   for w, m in zip(works, ms)]
            if need_first:
                firsts = tuple(jnp.where(k == 0, i0, fr)
                               for i0, fr in zip(i0s, firsts))
            works = tuple(jnp.where(gidx == i0, f32(-5.0), w)
                          for i0, w in zip(i0s, works))
            return works, firsts

        init = (tuple(rowlist), tuple(jnp.int32(0) for _ in rowlist))
        works, firsts = jax.lax.fori_loop(0, _K_NUM, body, init)
        founds = [w != r for w, r in zip(works, rowlist)]
        return founds, firsts

    killed = [jnp.zeros(shp, jnp.bool_) for _ in range(n)]
    maxrow = [jnp.full(shp, -1e30, f32) for _ in range(n)]
    loss_l = [f32(0.0)] * n
    loss_lm = [f32(0.0)] * n
    nval = [f32(0.0)] * n
    n1val = [f32(0.0)] * n

    for i in range(nobj):
        tx1 = [targ_ref[j, i, 0] for j in range(n)]
        ty1 = [targ_ref[j, i, 1] for j in range(n)]
        tx2 = [targ_ref[j, i, 2] for j in range(n)]
        ty2 = [targ_ref[j, i, 3] for j in range(n)]
        area_t = [(tx2[j] - tx1[j]) * (ty2[j] - ty1[j]) for j in range(n)]

        iou_a, iou_p, rowa = [], [], []
        for j in range(n):
            iw = jnp.clip(jnp.minimum(tx2[j], px2) - jnp.maximum(tx1[j], px1),
                          0.0, None)
            ih = jnp.clip(jnp.minimum(ty2[j], py2) - jnp.maximum(ty1[j], py1),
                          0.0, None)
            inter_a = iw * ih
            ia = inter_a / (area_t[j] + area_p - inter_a)
            iou_a.append(ia)

            iwp = jnp.clip(
                jnp.minimum(tx2[j], dx2[j]) - jnp.maximum(tx1[j], dx1[j]),
                0.0, None)
            ihp = jnp.clip(
                jnp.minimum(ty2[j], dy2[j]) - jnp.maximum(ty1[j], dy1[j]),
                0.0, None)
            inter_p = iwp * ihp
            ip = inter_p / (area_t[j] + area_d[j] - inter_p)
            iou_p.append(ip)

            maxrow[j] = jnp.maximum(maxrow[j], jnp.maximum(ia, ip))
            ra = jnp.where(killed[j], f32(-1.0), ia)
            rowa.append(jnp.where(validm, ra, f32(-3.0)))

        founda, firsta = topk_found(rowa, True)
        hit_a, rowp = [], []
        for j in range(n):
            passa = jnp.where(founda[j] & (rowa[j] > _P_TH), f32(1.0),
                              f32(0.0))
            anyg = jnp.max(passa) > 0
            firsta_f = jnp.where(gidx == firsta[j], f32(1.0), f32(0.0))
            ha = jnp.where(anyg, passa, firsta_f) > 0
            hit_a.append(ha)
            killed[j] = killed[j] | ha
            rp = jnp.where(killed[j], f32(-1.0), iou_p[j])
            rowp.append(jnp.where(validm, rp, f32(-3.0)))

        foundp, _ = topk_found(rowp, False)
        for j in range(n):
            hit_p = foundp[j] & (rowp[j] > _P_TH2)
            killed[j] = killed[j] | hit_p
            mi = hit_a[j] | hit_p

            g_cx = ((tx1[j] + tx2[j]) / 2.0 - pcx) / (_V0 * pw)
            g_cy = ((ty1[j] + ty2[j]) / 2.0 - pcy) / (_V0 * ph)
            g_w = jnp.log((tx2[j] - tx1[j]) / pw) / _V1
            g_h = jnp.log((ty2[j] - ty1[j]) / ph) / _V1
            lrow = (_sl1(jnp.abs(l0[j] - g_cx)) + _sl1(jnp.abs(l1[j] - g_cy))
                    + _sl1(jnp.abs(l2[j] - g_w)) + _sl1(jnp.abs(l3[j] - g_h)))
            loss_l[j] = loss_l[j] + jnp.sum(jnp.where(mi, lrow, f32(0.0)))

            lmrow = jnp.zeros(shp, f32)
            for p in range(3):
                lmx = targ_ref[j, i, 4 + 2 * p]
                lmy = targ_ref[j, i, 5 + 2 * p]
                gx = (lmx - pcx) / (_V0 * pw)
                gy = (lmy - pcy) / (_V0 * ph)
                lmrow = lmrow + _sl1(jnp.abs(landm_ref[j, 2 * p] - gx))
                lmrow = lmrow + _sl1(jnp.abs(landm_ref[j, 2 * p + 1] - gy))
            lab = targ_ref[j, i, 14]
            labf = jnp.where(lab == 1.0, f32(1.0), f32(0.0))
            loss_lm[j] = loss_lm[j] + labf * jnp.sum(
                jnp.where(mi, lmrow, f32(0.0)))

            cnt_i = jnp.sum(mi.astype(f32))
            nval[j] = nval[j] + cnt_i
            n1val[j] = n1val[j] + labf * cnt_i

    sel, dcf, negval, logz, t_eff, cnt_sel = [], [], [], [], [], []
    for j in range(n):
        sel.append(validm & (~killed[j]) & (maxrow[j] < _N_TH))
        c0 = conf_ref[j, 0]
        c1 = conf_ref[j, 1]
        m01 = jnp.maximum(c0, c1)
        lz = m01 + jnp.log(jnp.exp(c0 - m01) + jnp.exp(c1 - m01))
        logz.append(lz)
        dcf.append(c1 - c0)
        negval.append(lz - c0)
        num_pos = jnp.sum(jnp.where(killed[j], f32(1.0), f32(0.0)))
        cs = jnp.sum(jnp.where(sel[j], f32(1.0), f32(0.0)))
        cnt_sel.append(cs)
        t_eff.append(jnp.minimum(num_pos * _NEGPOS, cs))

    his = tuple(jnp.max(jnp.where(sel[j], dcf[j], f32(-1e30)))
                for j in range(n))
    los = tuple(jnp.min(jnp.where(sel[j], dcf[j], f32(1e30))) - 1.0
                for j in range(n))

    def bbody(_, carry):
        los, his = carry
        nlo, nhi = [], []
        for j in range(n):
            mid = 0.5 * (los[j] + his[j])
            c = jnp.sum(jnp.where(sel[j] & (dcf[j] > mid), f32(1.0), f32(0.0)))
            ge = c >= t_eff[j]
            nlo.append(jnp.where(ge, mid, los[j]))
            nhi.append(jnp.where(ge, his[j], mid))
        return tuple(nlo), tuple(nhi)

    los, his = jax.lax.fori_loop(0, _BISECT_ITERS, bbody, (los, his))

    for j in range(n):
        s_all = jnp.sum(jnp.where(sel[j], negval[j], f32(0.0)))
        above = sel[j] & (dcf[j] > his[j])
        g_cnt = jnp.sum(jnp.where(above, f32(1.0), f32(0.0)))
        s1 = jnp.sum(jnp.where(above, negval[j], f32(0.0)))
        bnd = sel[j] & (dcf[j] <= his[j]) & (dcf[j] > los[j])
        cnt2 = jnp.sum(jnp.where(bnd, f32(1.0), f32(0.0)))
        s2 = jnp.sum(jnp.where(bnd, negval[j], f32(0.0)))
        s_bis = s1 + (t_eff[j] - g_cnt) * (s2 / jnp.maximum(cnt2, f32(1.0)))
        s_neg = jnp.where(t_eff[j] >= cnt_sel[j], s_all, s_bis)
        pos_part = jnp.sum(jnp.where(killed[j], logz[j] - conf_ref[j, 1],
                                     f32(0.0)))
        loss_cls = pos_part + s_neg

        lane = jax.lax.broadcasted_iota(jnp.int32, (1, _LANES), 1)
        vec = jnp.where(
            lane == 0, loss_l[j],
            jnp.where(lane == 1, loss_cls,
                      jnp.where(lane == 2, loss_lm[j],
                                jnp.where(lane == 3, nval[j],
                                          jnp.where(lane == 4, n1val[j],
                                                    f32(0.0))))))
        out_ref[j] = vec


@jax.jit
def kernel(loc_data, conf_data, landm_data, priors, targets):
    num, num_anchor, _ = loc_data.shape
    nobj = targets.shape[1]
    rows = (num_anchor + _LANES - 1) // _LANES
    rows = ((rows + 7) // 8) * 8
    a_pad = rows * _LANES
    pad = a_pad - num_anchor

    def prep(x):
        x = jnp.pad(x, ((0, 0), (0, pad), (0, 0)))
        return x.transpose(0, 2, 1).reshape(num, x.shape[2], rows, _LANES)

    loc_r = prep(loc_data)
    conf_r = prep(conf_data)
    landm_r = prep(landm_data)
    pri_r = jnp.pad(priors, ((0, pad), (0, 0))).T.reshape(4, rows, _LANES)
    targ = jnp.pad(targets, ((0, 0), (0, 0), (0, 1)))

    body = functools.partial(_image_body, nobj, rows, num_anchor)
    out = pl.pallas_call(
        body,
        grid=(num // _NIMG,),
        in_specs=[
            pl.BlockSpec((_NIMG, 4, rows, _LANES), lambda b: (b, 0, 0, 0)),
            pl.BlockSpec((_NIMG, _NUM_CLASSES, rows, _LANES),
                         lambda b: (b, 0, 0, 0)),
            pl.BlockSpec((_NIMG, 6, rows, _LANES), lambda b: (b, 0, 0, 0)),
            pl.BlockSpec((4, rows, _LANES), lambda b: (0, 0, 0)),
            pl.BlockSpec((_NIMG, nobj, 16), lambda b: (b, 0, 0),
                         memory_space=pltpu.SMEM),
        ],
        out_specs=pl.BlockSpec((_NIMG, 1, _LANES), lambda b: (b, 0, 0)),
        out_shape=jax.ShapeDtypeStruct((num, 1, _LANES), jnp.float32),
    )(loc_r, conf_r, landm_r, pri_r, targ)

    sums = jnp.sum(out[:, 0, :5], axis=0)
    return (sums[0] / sums[3], sums[1] / sums[3], sums[2] / sums[4])


# speedup vs baseline: 19.7745x; 1.2286x over previous
import functools

import jax
import jax.numpy as jnp
import numpy as np
from jax.experimental import pallas as pl
from jax.experimental.pallas import tpu as pltpu

_NUM_CLASSES = 2
_P_TH = np.float32(0.35)
_P_TH2 = np.float32(0.35 + 0.05)
_N_TH = np.float32(0.35)
_K_NUM = 10
_NEGPOS = 7.0
_V0 = np.float32(0.1)
_V1 = np.float32(0.2)

_LANES = 128
_NIMG = 8
_BISECT_ITERS = 40


def _sl1(d):
    return jnp.where(d < 1.0, 0.5 * d * d, d - 0.5)


def _image_body(nobj, rows, num_anchor, loc_ref, conf_ref, landm_ref, pri_ref,
                targ_ref, out_ref):
    f32 = jnp.float32
    shp = (rows, _LANES)
    gidx = (jax.lax.broadcasted_iota(jnp.int32, shp, 0) * _LANES
            + jax.lax.broadcasted_iota(jnp.int32, shp, 1))
    validm = gidx < num_anchor
    big_idx = jnp.int32(rows * _LANES)

    pcx = pri_ref[0]
    pcy = pri_ref[1]
    pw = pri_ref[2]
    ph = pri_ref[3]
    px1 = pcx - pw / 2.0
    py1 = pcy - ph / 2.0
    px2 = pcx + pw / 2.0
    py2 = pcy + ph / 2.0
    area_p = (px2 - px1) * (py2 - py1)

    n = _NIMG
    l0 = [loc_ref[j, 0] for j in range(n)]
    l1 = [loc_ref[j, 1] for j in range(n)]
    l2 = [loc_ref[j, 2] for j in range(n)]
    l3 = [loc_ref[j, 3] for j in range(n)]
    dx1, dy1, dx2, dy2, area_d = [], [], [], [], []
    for j in range(n):
        dcx = pcx + l0[j] * _V0 * pw
        dcy = pcy + l1[j] * _V0 * ph
        dw = pw * jnp.exp(l2[j] * _V1)
        dh = ph * jnp.exp(l3[j] * _V1)
        dx1.append(dcx - dw / 2.0)
        dy1.append(dcy - dh / 2.0)
        dx2.append(dcx + dw / 2.0)
        dy2.append(dcy + dh / 2.0)
        area_d.append((dx2[j] - dx1[j]) * (dy2[j] - dy1[j]))

    def topk_found(rowlist, need_first):
        def body(k, carry):
            works, firsts = carry
            ms = [jnp.max(w) for w in works]
            i0s = [jnp.min(jnp.where(w == m, gidx, big_idx))
                   for w, m in zip(works, ms)]
            if need_first:
                firsts = tuple(jnp.where(k == 0, i0, fr)
                               for i0, fr in zip(i0s, firsts))
            works = tuple(jnp.where(gidx == i0, f32(-5.0), w)
                          for i0, w in zip(i0s, works))
            return works, firsts

        init = (tuple(rowlist), tuple(jnp.int32(0) for _ in rowlist))
        works, firsts = jax.lax.fori_loop(0, _K_NUM, body, init)
        founds = [w != r for w, r in zip(works, rowlist)]
        return founds, firsts

    killed = [jnp.zeros(shp, jnp.bool_) for _ in range(n)]
    maxrow = [jnp.full(shp, -1e30, f32) for _ in range(n)]
    loss_l = [f32(0.0)] * n
    loss_lm = [f32(0.0)] * n
    nval = [f32(0.0)] * n
    n1val = [f32(0.0)] * n

    for i in range(nobj):
        tx1 = [targ_ref[j, i, 0] for j in range(n)]
        ty1 = [targ_ref[j, i, 1] for j in range(n)]
        tx2 = [targ_ref[j, i, 2] for j in range(n)]
        ty2 = [targ_ref[j, i, 3] for j in range(n)]
        area_t = [(tx2[j] - tx1[j]) * (ty2[j] - ty1[j]) for j in range(n)]

        iou_a, iou_p, rowa = [], [], []
        for j in range(n):
            iw = jnp.clip(jnp.minimum(tx2[j], px2) - jnp.maximum(tx1[j], px1),
                          0.0, None)
            ih = jnp.clip(jnp.minimum(ty2[j], py2) - jnp.maximum(ty1[j], py1),
                          0.0, None)
            inter_a = iw * ih
            ia = inter_a / (area_t[j] + area_p - inter_a)
            iou_a.append(ia)

            iwp = jnp.clip(
                jnp.minimum(tx2[j], dx2[j]) - jnp.maximum(tx1[j], dx1[j]),
                0.0, None)
            ihp = jnp.clip(
                jnp.minimum(ty2[j], dy2[j]) - jnp.maximum(ty1[j], dy1[j]),
                0.0, None)
            inter_p = iwp * ihp
            ip = inter_p / (area_t[j] + area_d[j] - inter_p)
            iou_p.append(ip)

            maxrow[j] = jnp.maximum(maxrow[j], jnp.maximum(ia, ip))
            ra = jnp.where(killed[j], f32(-1.0), ia)
            rowa.append(jnp.where(validm, ra, f32(-3.0)))

        founda, firsta = topk_found(rowa, True)
        hit_a, rowp = [], []
        for j in range(n):
            passa = jnp.where(founda[j] & (rowa[j] > _P_TH), f32(1.0),
                              f32(0.0))
            anyg = jnp.max(passa) > 0
            firsta_f = jnp.where(gidx == firsta[j], f32(1.0), f32(0.0))
            ha = jnp.where(anyg, passa, firsta_f) > 0
            hit_a.append(ha)
            killed[j] = killed[j] | ha
            rp = jnp.where(killed[j], f32(-1.0), iou_p[j])
            rowp.append(jnp.where(validm, rp, f32(-3.0)))

        foundp, _ = topk_found(rowp, False)
        for j in range(n):
            hit_p = foundp[j] & (rowp[j] > _P_TH2)
            killed[j] = killed[j] | hit_p
            mi = hit_a[j] | hit_p

            g_cx = ((tx1[j] + tx2[j]) / 2.0 - pcx) / (_V0 * pw)
            g_cy = ((ty1[j] + ty2[j]) / 2.0 - pcy) / (_V0 * ph)
            g_w = jnp.log((tx2[j] - tx1[j]) / pw) / _V1
            g_h = jnp.log((ty2[j] - ty1[j]) / ph) / _V1
            lrow = (_sl1(jnp.abs(l0[j] - g_cx)) + _sl1(jnp.abs(l1[j] - g_cy))
                    + _sl1(jnp.abs(l2[j] - g_w)) + _sl1(jnp.abs(l3[j] - g_h)))
            loss_l[j] = loss_l[j] + jnp.sum(jnp.where(mi, lrow, f32(0.0)))

            lmrow = jnp.zeros(shp, f32)
            for p in range(3):
                lmx = targ_ref[j, i, 4 + 2 * p]
                lmy = targ_ref[j, i, 5 + 2 * p]
                gx = (lmx - pcx) / (_V0 * pw)
                gy = (lmy - pcy) / (_V0 * ph)
                lmrow = lmrow + _sl1(jnp.abs(landm_ref[j, 2 * p] - gx))
                lmrow = lmrow + _sl1(jnp.abs(landm_ref[j, 2 * p + 1] - gy))
            lab = targ_ref[j, i, 14]
            labf = jnp.where(lab == 1.0, f32(1.0), f32(0.0))
            loss_lm[j] = loss_lm[j] + labf * jnp.sum(
                jnp.where(mi, lmrow, f32(0.0)))

            cnt_i = jnp.sum(mi.astype(f32))
            nval[j] = nval[j] + cnt_i
            n1val[j] = n1val[j] + labf * cnt_i

    sel, dcf, negval, logz, t_eff, cnt_sel = [], [], [], [], [], []
    for j in range(n):
        sel.append(validm & (~killed[j]) & (maxrow[j] < _N_TH))
        c0 = conf_ref[j, 0]
        c1 = conf_ref[j, 1]
        m01 = jnp.maximum(c0, c1)
        lz = m01 + jnp.log(jnp.exp(c0 - m01) + jnp.exp(c1 - m01))
        logz.append(lz)
        dcf.append(c1 - c0)
        negval.append(lz - c0)
        num_pos = jnp.sum(jnp.where(killed[j], f32(1.0), f32(0.0)))
        cs = jnp.sum(jnp.where(sel[j], f32(1.0), f32(0.0)))
        cnt_sel.append(cs)
        t_eff.append(jnp.minimum(num_pos * _NEGPOS, cs))

    his = tuple(jnp.max(jnp.where(sel[j], dcf[j], f32(-1e30)))
                for j in range(n))
    los = tuple(jnp.min(jnp.where(sel[j], dcf[j], f32(1e30))) - 1.0
                for j in range(n))

    def bbody(_, carry):
        los, his = carry
        nlo, nhi = [], []
        for j in range(n):
            mid = 0.5 * (los[j] + his[j])
            c = jnp.sum(jnp.where(sel[j] & (dcf[j] > mid), f32(1.0), f32(0.0)))
            ge = c >= t_eff[j]
            nlo.append(jnp.where(ge, mid, los[j]))
            nhi.append(jnp.where(ge, his[j], mid))
        return tuple(nlo), tuple(nhi)

    los, his = jax.lax.fori_loop(0, _BISECT_ITERS, bbody, (los, his))

    for j in range(n):
        s_all = jnp.sum(jnp.where(sel[j], negval[j], f32(0.0)))
        above = sel[j] & (dcf[j] > his[j])
        g_cnt = jnp.sum(jnp.where(above, f32(1.0), f32(0.0)))
        s1 = jnp.sum(jnp.where(above, negval[j], f32(0.0)))
        bnd = sel[j] & (dcf[j] <= his[j]) & (dcf[j] > los[j])
        cnt2 = jnp.sum(jnp.where(bnd, f32(1.0), f32(0.0)))
        s2 = jnp.sum(jnp.where(bnd, negval[j], f32(0.0)))
        s_bis = s1 + (t_eff[j] - g_cnt) * (s2 / jnp.maximum(cnt2, f32(1.0)))
        s_neg = jnp.where(t_eff[j] >= cnt_sel[j], s_all, s_bis)
        pos_part = jnp.sum(jnp.where(killed[j], logz[j] - conf_ref[j, 1],
                                     f32(0.0)))
        loss_cls = pos_part + s_neg

        lane = jax.lax.broadcasted_iota(jnp.int32, (1, _LANES), 1)
        vec = jnp.where(
            lane == 0, loss_l[j],
            jnp.where(lane == 1, loss_cls,
                      jnp.where(lane == 2, loss_lm[j],
                                jnp.where(lane == 3, nval[j],
                                          jnp.where(lane == 4, n1val[j],
                                                    f32(0.0))))))
        out_ref[j] = vec


@jax.jit
def kernel(loc_data, conf_data, landm_data, priors, targets):
    num, num_anchor, _ = loc_data.shape
    nobj = targets.shape[1]
    rows = (num_anchor + _LANES - 1) // _LANES
    rows = ((rows + 7) // 8) * 8
    a_pad = rows * _LANES
    pad = a_pad - num_anchor

    def prep(x):
        x = jnp.pad(x, ((0, 0), (0, pad), (0, 0)))
        return x.transpose(0, 2, 1).reshape(num, x.shape[2], rows, _LANES)

    loc_r = prep(loc_data)
    conf_r = prep(conf_data)
    landm_r = prep(landm_data)
    pri_r = jnp.pad(priors, ((0, pad), (0, 0))).T.reshape(4, rows, _LANES)
    targ = jnp.pad(targets, ((0, 0), (0, 0), (0, 1)))

    body = functools.partial(_image_body, nobj, rows, num_anchor)
    out = pl.pallas_call(
        body,
        grid=(num // _NIMG,),
        in_specs=[
            pl.BlockSpec((_NIMG, 4, rows, _LANES), lambda b: (b, 0, 0, 0)),
            pl.BlockSpec((_NIMG, _NUM_CLASSES, rows, _LANES),
                         lambda b: (b, 0, 0, 0)),
            pl.BlockSpec((_NIMG, 6, rows, _LANES), lambda b: (b, 0, 0, 0)),
            pl.BlockSpec((4, rows, _LANES), lambda b: (0, 0, 0)),
            pl.BlockSpec((_NIMG, nobj, 16), lambda b: (b, 0, 0),
                         memory_space=pltpu.SMEM),
        ],
        out_specs=pl.BlockSpec((_NIMG, 1, _LANES), lambda b: (b, 0, 0)),
        out_shape=jax.ShapeDtypeStruct((num, 1, _LANES), jnp.float32),
    )(loc_r, conf_r, landm_r, pri_r, targ)

    sums = jnp.sum(out[:, 0, :5], axis=0)
    return (sums[0] / sums[3], sums[1] / sums[3], sums[2] / sums[4])


# speedup vs baseline: 25.6856x; 1.2989x over previous
import functools

import jax
import jax.numpy as jnp
import numpy as np
from jax.experimental import pallas as pl
from jax.experimental.pallas import tpu as pltpu

_NUM_CLASSES = 2
_P_TH = np.float32(0.35)
_P_TH2 = np.float32(0.35 + 0.05)
_N_TH = np.float32(0.35)
_K_NUM = 10
_NEGPOS = 7.0
_V0 = np.float32(0.1)
_V1 = np.float32(0.2)

_LANES = 128
_NIMG = 16
_BISECT_ITERS = 40


def _sl1(d):
    return jnp.where(d < 1.0, 0.5 * d * d, d - 0.5)


def _image_body(nobj, rows, num_anchor, loc_ref, conf_ref, landm_ref, pri_ref,
                targ_ref, out_ref):
    f32 = jnp.float32
    shp = (rows, _LANES)
    gidx = (jax.lax.broadcasted_iota(jnp.int32, shp, 0) * _LANES
            + jax.lax.broadcasted_iota(jnp.int32, shp, 1))
    validm = gidx < num_anchor
    big_idx = jnp.int32(rows * _LANES)

    pcx = pri_ref[0]
    pcy = pri_ref[1]
    pw = pri_ref[2]
    ph = pri_ref[3]
    px1 = pcx - pw / 2.0
    py1 = pcy - ph / 2.0
    px2 = pcx + pw / 2.0
    py2 = pcy + ph / 2.0
    area_p = (px2 - px1) * (py2 - py1)

    n = _NIMG
    l0 = [loc_ref[j, 0] for j in range(n)]
    l1 = [loc_ref[j, 1] for j in range(n)]
    l2 = [loc_ref[j, 2] for j in range(n)]
    l3 = [loc_ref[j, 3] for j in range(n)]
    dx1, dy1, dx2, dy2, area_d = [], [], [], [], []
    for j in range(n):
        dcx = pcx + l0[j] * _V0 * pw
        dcy = pcy + l1[j] * _V0 * ph
        dw = pw * jnp.exp(l2[j] * _V1)
        dh = ph * jnp.exp(l3[j] * _V1)
        dx1.append(dcx - dw / 2.0)
        dy1.append(dcy - dh / 2.0)
        dx2.append(dcx + dw / 2.0)
        dy2.append(dcy + dh / 2.0)
        area_d.append((dx2[j] - dx1[j]) * (dy2[j] - dy1[j]))

    def topk_found(rowlist, need_first):
        def body(k, carry):
            works, firsts = carry
            ms = [jnp.max(w) for w in works]
            i0s = [jnp.min(jnp.where(w == m, gidx, big_idx))
                   for w, m in zip(works, ms)]
            if need_first:
                firsts = tuple(jnp.where(k == 0, i0, fr)
                               for i0, fr in zip(i0s, firsts))
            works = tuple(jnp.where(gidx == i0, f32(-5.0), w)
                          for i0, w in zip(i0s, works))
            return works, firsts

        init = (tuple(rowlist), tuple(jnp.int32(0) for _ in rowlist))
        works, firsts = jax.lax.fori_loop(0, _K_NUM, body, init)
        founds = [w != r for w, r in zip(works, rowlist)]
        return founds, firsts

    killed = [jnp.zeros(shp, jnp.bool_) for _ in range(n)]
    maxrow = [jnp.full(shp, -1e30, f32) for _ in range(n)]
    loss_l = [f32(0.0)] * n
    loss_lm = [f32(0.0)] * n
    nval = [f32(0.0)] * n
    n1val = [f32(0.0)] * n

    for i in range(nobj):
        tx1 = [targ_ref[j, i, 0] for j in range(n)]
        ty1 = [targ_ref[j, i, 1] for j in range(n)]
        tx2 = [targ_ref[j, i, 2] for j in range(n)]
        ty2 = [targ_ref[j, i, 3] for j in range(n)]
        area_t = [(tx2[j] - tx1[j]) * (ty2[j] - ty1[j]) for j in range(n)]

        iou_a, iou_p, rowa = [], [], []
        for j in range(n):
            iw = jnp.clip(jnp.minimum(tx2[j], px2) - jnp.maximum(tx1[j], px1),
                          0.0, None)
            ih = jnp.clip(jnp.minimum(ty2[j], py2) - jnp.maximum(ty1[j], py1),
                          0.0, None)
            inter_a = iw * ih
            ia = inter_a / (area_t[j] + area_p - inter_a)
            iou_a.append(ia)

            iwp = jnp.clip(
                jnp.minimum(tx2[j], dx2[j]) - jnp.maximum(tx1[j], dx1[j]),
                0.0, None)
            ihp = jnp.clip(
                jnp.minimum(ty2[j], dy2[j]) - jnp.maximum(ty1[j], dy1[j]),
                0.0, None)
            inter_p = iwp * ihp
            ip = inter_p / (area_t[j] + area_d[j] - inter_p)
            iou_p.append(ip)

            maxrow[j] = jnp.maximum(maxrow[j], jnp.maximum(ia, ip))
            ra = jnp.where(killed[j], f32(-1.0), ia)
            rowa.append(jnp.where(validm, ra, f32(-3.0)))

        founda, firsta = topk_found(rowa, True)
        hit_a, rowp = [], []
        for j in range(n):
            passa = jnp.where(founda[j] & (rowa[j] > _P_TH), f32(1.0),
                              f32(0.0))
            anyg = jnp.max(passa) > 0
            firsta_f = jnp.where(gidx == firsta[j], f32(1.0), f32(0.0))
            ha = jnp.where(anyg, passa, firsta_f) > 0
            hit_a.append(ha)
            killed[j] = killed[j] | ha
            rp = jnp.where(killed[j], f32(-1.0), iou_p[j])
            rowp.append(jnp.where(validm, rp, f32(-3.0)))

        foundp, _ = topk_found(rowp, False)
        for j in range(n):
            hit_p = foundp[j] & (rowp[j] > _P_TH2)
            killed[j] = killed[j] | hit_p
            mi = hit_a[j] | hit_p

            g_cx = ((tx1[j] + tx2[j]) / 2.0 - pcx) / (_V0 * pw)
            g_cy = ((ty1[j] + ty2[j]) / 2.0 - pcy) / (_V0 * ph)
            g_w = jnp.log((tx2[j] - tx1[j]) / pw) / _V1
            g_h = jnp.log((ty2[j] - ty1[j]) / ph) / _V1
            lrow = (_sl1(jnp.abs(l0[j] - g_cx)) + _sl1(jnp.abs(l1[j] - g_cy))
                    + _sl1(jnp.abs(l2[j] - g_w)) + _sl1(jnp.abs(l3[j] - g_h)))
            loss_l[j] = loss_l[j] + jnp.sum(jnp.where(mi, lrow, f32(0.0)))

            lmrow = jnp.zeros(shp, f32)
            for p in range(3):
                lmx = targ_ref[j, i, 4 + 2 * p]
                lmy = targ_ref[j, i, 5 + 2 * p]
                gx = (lmx - pcx) / (_V0 * pw)
                gy = (lmy - pcy) / (_V0 * ph)
                lmrow = lmrow + _sl1(jnp.abs(landm_ref[j, 2 * p] - gx))
                lmrow = lmrow + _sl1(jnp.abs(landm_ref[j, 2 * p + 1] - gy))
            lab = targ_ref[j, i, 14]
            labf = jnp.where(lab == 1.0, f32(1.0), f32(0.0))
            loss_lm[j] = loss_lm[j] + labf * jnp.sum(
                jnp.where(mi, lmrow, f32(0.0)))

            cnt_i = jnp.sum(mi.astype(f32))
            nval[j] = nval[j] + cnt_i
            n1val[j] = n1val[j] + labf * cnt_i

    sel, dcf, negval, logz, t_eff, cnt_sel = [], [], [], [], [], []
    for j in range(n):
        sel.append(validm & (~killed[j]) & (maxrow[j] < _N_TH))
        c0 = conf_ref[j, 0]
        c1 = conf_ref[j, 1]
        m01 = jnp.maximum(c0, c1)
        lz = m01 + jnp.log(jnp.exp(c0 - m01) + jnp.exp(c1 - m01))
        logz.append(lz)
        dcf.append(c1 - c0)
        negval.append(lz - c0)
        num_pos = jnp.sum(jnp.where(killed[j], f32(1.0), f32(0.0)))
        cs = jnp.sum(jnp.where(sel[j], f32(1.0), f32(0.0)))
        cnt_sel.append(cs)
        t_eff.append(jnp.minimum(num_pos * _NEGPOS, cs))

    his = tuple(jnp.max(jnp.where(sel[j], dcf[j], f32(-1e30)))
                for j in range(n))
    los = tuple(jnp.min(jnp.where(sel[j], dcf[j], f32(1e30))) - 1.0
                for j in range(n))

    def bbody(_, carry):
        los, his = carry
        nlo, nhi = [], []
        for j in range(n):
            mid = 0.5 * (los[j] + his[j])
            c = jnp.sum(jnp.where(sel[j] & (dcf[j] > mid), f32(1.0), f32(0.0)))
            ge = c >= t_eff[j]
            nlo.append(jnp.where(ge, mid, los[j]))
            nhi.append(jnp.where(ge, his[j], mid))
        return tuple(nlo), tuple(nhi)

    los, his = jax.lax.fori_loop(0, _BISECT_ITERS, bbody, (los, his))

    for j in range(n):
        s_all = jnp.sum(jnp.where(sel[j], negval[j], f32(0.0)))
        above = sel[j] & (dcf[j] > his[j])
        g_cnt = jnp.sum(jnp.where(above, f32(1.0), f32(0.0)))
        s1 = jnp.sum(jnp.where(above, negval[j], f32(0.0)))
        bnd = sel[j] & (dcf[j] <= his[j]) & (dcf[j] > los[j])
        cnt2 = jnp.sum(jnp.where(bnd, f32(1.0), f32(0.0)))
        s2 = jnp.sum(jnp.where(bnd, negval[j], f32(0.0)))
        s_bis = s1 + (t_eff[j] - g_cnt) * (s2 / jnp.maximum(cnt2, f32(1.0)))
        s_neg = jnp.where(t_eff[j] >= cnt_sel[j], s_all, s_bis)
        pos_part = jnp.sum(jnp.where(killed[j], logz[j] - conf_ref[j, 1],
                                     f32(0.0)))
        loss_cls = pos_part + s_neg

        lane = jax.lax.broadcasted_iota(jnp.int32, (1, _LANES), 1)
        vec = jnp.where(
            lane == 0, loss_l[j],
            jnp.where(lane == 1, loss_cls,
                      jnp.where(lane == 2, loss_lm[j],
                                jnp.where(lane == 3, nval[j],
                                          jnp.where(lane == 4, n1val[j],
                                                    f32(0.0))))))
        out_ref[j] = vec


@jax.jit
def kernel(loc_data, conf_data, landm_data, priors, targets):
    num, num_anchor, _ = loc_data.shape
    nobj = targets.shape[1]
    rows = (num_anchor + _LANES - 1) // _LANES
    rows = ((rows + 7) // 8) * 8
    a_pad = rows * _LANES
    pad = a_pad - num_anchor

    def prep(x):
        x = jnp.pad(x, ((0, 0), (0, pad), (0, 0)))
        return x.transpose(0, 2, 1).reshape(num, x.shape[2], rows, _LANES)

    loc_r = prep(loc_data)
    conf_r = prep(conf_data)
    landm_r = prep(landm_data)
    pri_r = jnp.pad(priors, ((0, pad), (0, 0))).T.reshape(4, rows, _LANES)
    targ = jnp.pad(targets, ((0, 0), (0, 0), (0, 1)))

    body = functools.partial(_image_body, nobj, rows, num_anchor)
    out = pl.pallas_call(
        body,
        grid=(num // _NIMG,),
        in_specs=[
            pl.BlockSpec((_NIMG, 4, rows, _LANES), lambda b: (b, 0, 0, 0)),
            pl.BlockSpec((_NIMG, _NUM_CLASSES, rows, _LANES),
                         lambda b: (b, 0, 0, 0)),
            pl.BlockSpec((_NIMG, 6, rows, _LANES), lambda b: (b, 0, 0, 0)),
            pl.BlockSpec((4, rows, _LANES), lambda b: (0, 0, 0)),
            pl.BlockSpec((_NIMG, nobj, 16), lambda b: (b, 0, 0),
                         memory_space=pltpu.SMEM),
        ],
        out_specs=pl.BlockSpec((_NIMG, 1, _LANES), lambda b: (b, 0, 0)),
        out_shape=jax.ShapeDtypeStruct((num, 1, _LANES), jnp.float32),
    )(loc_r, conf_r, landm_r, pri_r, targ)

    sums = jnp.sum(out[:, 0, :5], axis=0)
    return (sums[0] / sums[3], sums[1] / sums[3], sums[2] / sums[4])


# speedup vs baseline: 26.1736x; 1.0190x over previous
import functools

import jax
import jax.numpy as jnp
import numpy as np
from jax.experimental import pallas as pl
from jax.experimental.pallas import tpu as pltpu

_NUM_CLASSES = 2
_P_TH = np.float32(0.35)
_P_TH2 = np.float32(0.35 + 0.05)
_N_TH = np.float32(0.35)
_K_NUM = 10
_NEGPOS = 7.0
_V0 = np.float32(0.1)
_V1 = np.float32(0.2)

_LANES = 128
_NIMG = 16
_BISECT_ITERS = 34


def _sl1(d):
    return jnp.where(d < 1.0, 0.5 * d * d, d - 0.5)


def _image_body(nobj, rows, num_anchor, loc_ref, conf_ref, landm_ref, pri_ref,
                targ_ref, out_ref):
    f32 = jnp.float32
    shp = (rows, _LANES)
    gidx = (jax.lax.broadcasted_iota(jnp.int32, shp, 0) * _LANES
            + jax.lax.broadcasted_iota(jnp.int32, shp, 1))
    validm = gidx < num_anchor
    big_idx = jnp.int32(rows * _LANES)

    pcx = pri_ref[0]
    pcy = pri_ref[1]
    pw = pri_ref[2]
    ph = pri_ref[3]
    px1 = pcx - pw / 2.0
    py1 = pcy - ph / 2.0
    px2 = pcx + pw / 2.0
    py2 = pcy + ph / 2.0
    area_p = (px2 - px1) * (py2 - py1)
    i01pw = 1.0 / (_V0 * pw)
    i01ph = 1.0 / (_V0 * ph)
    logpw = jnp.log(pw)
    logph = jnp.log(ph)
    inv_v1 = f32(1.0 / 0.2)

    n = _NIMG
    l0 = [loc_ref[j, 0] for j in range(n)]
    l1 = [loc_ref[j, 1] for j in range(n)]
    l2 = [loc_ref[j, 2] for j in range(n)]
    l3 = [loc_ref[j, 3] for j in range(n)]
    dx1, dy1, dx2, dy2, area_d = [], [], [], [], []
    for j in range(n):
        dcx = pcx + l0[j] * _V0 * pw
        dcy = pcy + l1[j] * _V0 * ph
        dw = pw * jnp.exp(l2[j] * _V1)
        dh = ph * jnp.exp(l3[j] * _V1)
        dx1.append(dcx - dw / 2.0)
        dy1.append(dcy - dh / 2.0)
        dx2.append(dcx + dw / 2.0)
        dy2.append(dcy + dh / 2.0)
        area_d.append((dx2[j] - dx1[j]) * (dy2[j] - dy1[j]))

    def topk_found(rowlist, need_first):
        def body(k, carry):
            works, firsts = carry
            ms = [jnp.max(w) for w in works]
            i0s = [jnp.min(jnp.where(w == m, gidx, big_idx))
                   for w, m in zip(works, ms)]
            if need_first:
                firsts = tuple(jnp.where(k == 0, i0, fr)
                               for i0, fr in zip(i0s, firsts))
            works = tuple(jnp.where(gidx == i0, f32(-5.0), w)
                          for i0, w in zip(i0s, works))
            return works, firsts

        init = (tuple(rowlist), tuple(jnp.int32(0) for _ in rowlist))
        works, firsts = jax.lax.fori_loop(0, _K_NUM, body, init)
        founds = [w != r for w, r in zip(works, rowlist)]
        return founds, firsts

    notvalid = ~validm
    killed = [notvalid for _ in range(n)]
    maxrow = [jnp.full(shp, -1e30, f32) for _ in range(n)]
    loss_l = [f32(0.0)] * n
    loss_lm = [f32(0.0)] * n
    nval = [f32(0.0)] * n
    n1val = [f32(0.0)] * n

    for i in range(nobj):
        tx1 = [targ_ref[j, i, 0] for j in range(n)]
        ty1 = [targ_ref[j, i, 1] for j in range(n)]
        tx2 = [targ_ref[j, i, 2] for j in range(n)]
        ty2 = [targ_ref[j, i, 3] for j in range(n)]
        area_t = [(tx2[j] - tx1[j]) * (ty2[j] - ty1[j]) for j in range(n)]

        iou_a, iou_p, rowa = [], [], []
        for j in range(n):
            iw = jnp.clip(jnp.minimum(tx2[j], px2) - jnp.maximum(tx1[j], px1),
                          0.0, None)
            ih = jnp.clip(jnp.minimum(ty2[j], py2) - jnp.maximum(ty1[j], py1),
                          0.0, None)
            inter_a = iw * ih
            ia = inter_a / (area_t[j] + area_p - inter_a)
            iou_a.append(ia)

            iwp = jnp.clip(
                jnp.minimum(tx2[j], dx2[j]) - jnp.maximum(tx1[j], dx1[j]),
                0.0, None)
            ihp = jnp.clip(
                jnp.minimum(ty2[j], dy2[j]) - jnp.maximum(ty1[j], dy1[j]),
                0.0, None)
            inter_p = iwp * ihp
            ip = inter_p / (area_t[j] + area_d[j] - inter_p)
            iou_p.append(ip)

            maxrow[j] = jnp.maximum(maxrow[j], jnp.maximum(ia, ip))
            rowa.append(jnp.where(killed[j], f32(-1.0), ia))

        founda, firsta = topk_found(rowa, True)
        hit_a, rowp = [], []
        for j in range(n):
            passa = jnp.where(founda[j] & (rowa[j] > _P_TH), f32(1.0),
                              f32(0.0))
            anyg = jnp.max(passa) > 0
            firsta_f = jnp.where(gidx == firsta[j], f32(1.0), f32(0.0))
            ha = jnp.where(anyg, passa, firsta_f) > 0
            hit_a.append(ha)
            killed[j] = killed[j] | ha
            rowp.append(jnp.where(killed[j], f32(-1.0), iou_p[j]))

        foundp, _ = topk_found(rowp, False)
        for j in range(n):
            hit_p = foundp[j] & (rowp[j] > _P_TH2)
            killed[j] = killed[j] | hit_p
            mi = hit_a[j] | hit_p

            g_cx = (targ_ref[j, i, 15] - pcx) * i01pw
            g_cy = (targ_ref[j, i, 16] - pcy) * i01ph
            g_w = (targ_ref[j, i, 17] - logpw) * inv_v1
            g_h = (targ_ref[j, i, 18] - logph) * inv_v1
            lrow = (_sl1(jnp.abs(l0[j] - g_cx)) + _sl1(jnp.abs(l1[j] - g_cy))
                    + _sl1(jnp.abs(l2[j] - g_w)) + _sl1(jnp.abs(l3[j] - g_h)))
            loss_l[j] = loss_l[j] + jnp.sum(jnp.where(mi, lrow, f32(0.0)))

            lmrow = jnp.zeros(shp, f32)
            for p in range(3):
                lmx = targ_ref[j, i, 4 + 2 * p]
                lmy = targ_ref[j, i, 5 + 2 * p]
                gx = (lmx - pcx) * i01pw
                gy = (lmy - pcy) * i01ph
                lmrow = lmrow + _sl1(jnp.abs(landm_ref[j, 2 * p] - gx))
                lmrow = lmrow + _sl1(jnp.abs(landm_ref[j, 2 * p + 1] - gy))
            lab = targ_ref[j, i, 14]
            labf = jnp.where(lab == 1.0, f32(1.0), f32(0.0))
            loss_lm[j] = loss_lm[j] + labf * jnp.sum(
                jnp.where(mi, lmrow, f32(0.0)))

            cnt_i = jnp.sum(mi.astype(f32))
            nval[j] = nval[j] + cnt_i
            n1val[j] = n1val[j] + labf * cnt_i

    sel, dmask, dcf, negval, logz, t_eff, cnt_sel = [], [], [], [], [], [], []
    for j in range(n):
        sel.append((~killed[j]) & (maxrow[j] < _N_TH))
        c0 = conf_ref[j, 0]
        c1 = conf_ref[j, 1]
        m01 = jnp.maximum(c0, c1)
        lz = m01 + jnp.log(jnp.exp(c0 - m01) + jnp.exp(c1 - m01))
        logz.append(lz)
        d = c1 - c0
        dcf.append(d)
        dmask.append(jnp.where(sel[j], d, f32(-1e30)))
        negval.append(lz - c0)
        num_pos = nval[j]
        cs = jnp.sum(jnp.where(sel[j], f32(1.0), f32(0.0)))
        cnt_sel.append(cs)
        t_eff.append(jnp.minimum(num_pos * _NEGPOS, cs))

    his = tuple(jnp.max(dmask[j]) for j in range(n))
    los = tuple(jnp.min(jnp.where(sel[j], dcf[j], f32(1e30))) - 1.0
                for j in range(n))

    def bbody(_, carry):
        los, his = carry
        nlo, nhi = [], []
        for j in range(n):
            mid = 0.5 * (los[j] + his[j])
            c = jnp.sum(jnp.where(dmask[j] > mid, f32(1.0), f32(0.0)))
            ge = c >= t_eff[j]
            nlo.append(jnp.where(ge, mid, los[j]))
            nhi.append(jnp.where(ge, his[j], mid))
        return tuple(nlo), tuple(nhi)

    los, his = jax.lax.fori_loop(0, _BISECT_ITERS, bbody, (los, his))

    for j in range(n):
        s_all = jnp.sum(jnp.where(sel[j], negval[j], f32(0.0)))
        above = dmask[j] > his[j]
        g_cnt = jnp.sum(jnp.where(above, f32(1.0), f32(0.0)))
        s1 = jnp.sum(jnp.where(above, negval[j], f32(0.0)))
        bnd = (dmask[j] <= his[j]) & (dmask[j] > los[j])
        cnt2 = jnp.sum(jnp.where(bnd, f32(1.0), f32(0.0)))
        s2 = jnp.sum(jnp.where(bnd, negval[j], f32(0.0)))
        s_bis = s1 + (t_eff[j] - g_cnt) * (s2 / jnp.maximum(cnt2, f32(1.0)))
        s_neg = jnp.where(t_eff[j] >= cnt_sel[j], s_all, s_bis)
        pos_part = jnp.sum(jnp.where(killed[j] & validm,
                                     logz[j] - conf_ref[j, 1], f32(0.0)))
        loss_cls = pos_part + s_neg

        lane = jax.lax.broadcasted_iota(jnp.int32, (1, _LANES), 1)
        vec = jnp.where(
            lane == 0, loss_l[j],
            jnp.where(lane == 1, loss_cls,
                      jnp.where(lane == 2, loss_lm[j],
                                jnp.where(lane == 3, nval[j],
                                          jnp.where(lane == 4, n1val[j],
                                                    f32(0.0))))))
        out_ref[j] = vec


@jax.jit
def kernel(loc_data, conf_data, landm_data, priors, targets):
    num, num_anchor, _ = loc_data.shape
    nobj = targets.shape[1]
    rows = (num_anchor + _LANES - 1) // _LANES
    rows = ((rows + 7) // 8) * 8
    a_pad = rows * _LANES
    pad = a_pad - num_anchor

    def prep(x):
        x = jnp.pad(x, ((0, 0), (0, pad), (0, 0)))
        return x.transpose(0, 2, 1).reshape(num, x.shape[2], rows, _LANES)

    loc_r = prep(loc_data)
    conf_r = prep(conf_data)
    landm_r = prep(landm_data)
    pri_r = jnp.pad(priors, ((0, pad), (0, 0))).T.reshape(4, rows, _LANES)
    extras = jnp.stack(
        [(targets[..., 0] + targets[..., 2]) / 2.0,
         (targets[..., 1] + targets[..., 3]) / 2.0,
         jnp.log(targets[..., 2] - targets[..., 0]),
         jnp.log(targets[..., 3] - targets[..., 1])], axis=-1)
    targ = jnp.concatenate(
        [targets, extras,
         jnp.zeros((num, nobj, 5), jnp.float32)], axis=-1)

    body = functools.partial(_image_body, nobj, rows, num_anchor)
    out = pl.pallas_call(
        body,
        grid=(num // _NIMG,),
        in_specs=[
            pl.BlockSpec((_NIMG, 4, rows, _LANES), lambda b: (b, 0, 0, 0)),
            pl.BlockSpec((_NIMG, _NUM_CLASSES, rows, _LANES),
                         lambda b: (b, 0, 0, 0)),
            pl.BlockSpec((_NIMG, 6, rows, _LANES), lambda b: (b, 0, 0, 0)),
            pl.BlockSpec((4, rows, _LANES), lambda b: (0, 0, 0)),
            pl.BlockSpec((_NIMG, nobj, 24), lambda b: (b, 0, 0),
                         memory_space=pltpu.SMEM),
        ],
        out_specs=pl.BlockSpec((_NIMG, 1, _LANES), lambda b: (b, 0, 0)),
        out_shape=jax.ShapeDtypeStruct((num, 1, _LANES), jnp.float32),
    )(loc_r, conf_r, landm_r, pri_r, targ)

    sums = jnp.sum(out[:, 0, :5], axis=0)
    return (sums[0] / sums[3], sums[1] / sums[3], sums[2] / sums[4])
